# baseline (speedup 1.0000x reference)
import functools

import jax
import jax.numpy as jnp
from jax import lax
from jax.experimental import pallas as pl
from jax.experimental.pallas import tpu as pltpu
from jax.experimental.pallas import tpu_sc as plsc

_NC, _NS, _L = 2, 16, 16
_NW = _NC * _NS
_R = 16
_U = 8


def kernel(x, embed_weight):
    B, T, E = x.shape
    rows_w = T // _NW
    n_chunks = rows_w // _R

    mesh = plsc.VectorSubcoreMesh(core_axis_name="c", subcore_axis_name="s")

    @functools.partial(
        pl.kernel,
        mesh=mesh,
        out_type=jax.ShapeDtypeStruct((B, B, T, E), jnp.float32),
        scratch_types=[
            pltpu.VMEM((2, _R, E), jnp.float32),
            pltpu.VMEM((2, _R, E), jnp.float32),
            pltpu.VMEM((2, _R, E), jnp.float32),
            pltpu.SemaphoreType.DMA,
            pltpu.SemaphoreType.DMA,
            pltpu.SemaphoreType.DMA,
            pltpu.SemaphoreType.DMA,
        ],
        compiler_params=pltpu.CompilerParams(use_tc_tiling_on_sc=True),
    )
    def sc_add(x_hbm, w_hbm, out_hbm, b0, b1, bw, si0, si1, so0, so1):
        wid = lax.axis_index("s") * _NC + lax.axis_index("c")
        row0 = wid * rows_w
        sem_in = (si0, si1)
        sem_out = (so0, so1)

        def start_in(c):
            s = c % 2
            rows = pl.ds(row0 + c * _R, _R)
            return [
                pltpu.async_copy(x_hbm.at[0, rows], b0.at[s], sem_in[s]),
                pltpu.async_copy(x_hbm.at[1, rows], b1.at[s], sem_in[s]),
                pltpu.async_copy(w_hbm.at[rows], bw.at[s], sem_in[s]),
            ]

        def start_out(c):
            s = c % 2
            rows = pl.ds(row0 + c * _R, _R)
            cps = []
            for i in range(B):
                for j in range(B):
                    src = b0.at[s] if j == 0 else b1.at[s]
                    cps.append(
                        pltpu.async_copy(src, out_hbm.at[i, j, rows], sem_out[s]))
            return cps

        def compute(c):
            s = c % 2

            def row_body(r, carry):
                def vec_body(i, carry2):
                    off = i * (_L * _U)
                    for u in range(_U):
                        sl = pl.ds(off + u * _L, _L)
                        wv = bw[s, r, sl]
                        b0[s, r, sl] = b0[s, r, sl] + wv
                        b1[s, r, sl] = b1[s, r, sl] + wv
                    return carry2

                return lax.fori_loop(0, E // (_L * _U), vec_body, carry)

            lax.fori_loop(0, _R, row_body, 0)

        in_cps = {0: start_in(0)}
        out_cps = {}
        for c in range(n_chunks):
            for cp in in_cps.pop(c):
                cp.wait()
            if c + 1 < n_chunks:
                for cp in out_cps.pop(c - 1, ()):
                    cp.wait()
                in_cps[c + 1] = start_in(c + 1)
            compute(c)
            out_cps[c] = start_out(c)
        for c, cps in sorted(out_cps.items()):
            for cp in cps:
                cp.wait()

    return sc_add(x, embed_weight)


# speedup vs baseline: 4.2156x; 4.2156x over previous
import jax
import jax.numpy as jnp
from jax.experimental import pallas as pl


_TILE_T = 1024


def _add_pe_kernel(x_ref, w_ref, out_ref):
    w = w_ref[...]
    s0 = x_ref[0] + w
    s1 = x_ref[1] + w
    out_ref[0, 0] = s0
    out_ref[0, 1] = s1
    out_ref[1, 0] = s0
    out_ref[1, 1] = s1


def kernel(x, embed_weight):
    B, T, E = x.shape
    tile_t = min(_TILE_T, T)
    grid = (T // tile_t,)
    return pl.pallas_call(
        _add_pe_kernel,
        grid=grid,
        in_specs=[
            pl.BlockSpec((B, tile_t, E), lambda t: (0, t, 0)),
            pl.BlockSpec((tile_t, E), lambda t: (t, 0)),
        ],
        out_specs=pl.BlockSpec((B, B, tile_t, E), lambda t: (0, 0, t, 0)),
        out_shape=jax.ShapeDtypeStruct((B, B, T, E), x.dtype),
    )(x, embed_weight)


# speedup vs baseline: 4.2218x; 1.0015x over previous
import jax
import jax.numpy as jnp
from jax.experimental import pallas as pl


_TILE_T = 1024


def kernel(x, embed_weight):
    B, T, E = x.shape
    tile_t = min(_TILE_T, T)
    grid = (T // tile_t,)

    def _add_pe_kernel(x_ref, w_ref, out_ref):
        w = w_ref[...]
        for j in range(B):
            s = x_ref[j] + w
            for i in range(B):
                out_ref[i, j] = s

    return pl.pallas_call(
        _add_pe_kernel,
        grid=grid,
        in_specs=[
            pl.BlockSpec((B, tile_t, E), lambda t: (0, t, 0)),
            pl.BlockSpec((tile_t, E), lambda t: (t, 0)),
        ],
        out_specs=pl.BlockSpec((B, B, tile_t, E), lambda t: (0, 0, t, 0)),
        out_shape=jax.ShapeDtypeStruct((B, B, T, E), x.dtype),
    )(x, embed_weight)
